# Initial kernel scaffold; baseline (speedup 1.0000x reference)
import jax
import jax.numpy as jnp
from jax.experimental import pallas as pl


def kernel(x, edge_index, edge_attr, W_msg, b_msg, W_upd, b_upd, ln_gamma, ln_beta):
    raise NotImplementedError("write your pallas kernel here")


# speedup vs baseline: 2.0063x; 2.0063x over previous
import jax
import jax.numpy as jnp
from jax import lax
from jax.experimental import pallas as pl
from jax.experimental.pallas import tpu as pltpu
from jax.experimental.pallas import tpu_sc as plsc

N_NODES = 10000
N_EDGES = 320000
D = 128
DH = 64
D_EDGE = 16

NC = 2
NS = 16
CH = 64
N_CHUNKS = N_EDGES // CH
NH = 5120
ACC_ROWS = 5248
ROWS_PER_TILE = ACC_ROWS // NS
N_PAD = 10240


def _node_mm_body(x_ref, w_ref, ab_ref):
    ab_ref[...] = jnp.dot(x_ref[...], w_ref[0],
                          preferred_element_type=jnp.float32)


def _node_mm(x, w1, w2):
    bn = 1000
    nblk = N_NODES // bn
    w12 = jnp.stack([w1, w2])
    return pl.pallas_call(
        _node_mm_body,
        grid=(2 * nblk,),
        in_specs=[
            pl.BlockSpec((bn, D), lambda i: (i % nblk, 0)),
            pl.BlockSpec((1, D, D), lambda i: (i // nblk, 0, 0)),
        ],
        out_specs=pl.BlockSpec((bn, D), lambda i: (i, 0)),
        out_shape=jax.ShapeDtypeStruct((2 * N_NODES, D), jnp.float32),
    )(x, w12)


def _edge_mm_body(e_ref, w3_ref, b_ref, c_ref):
    c_ref[...] = (
        jnp.dot(e_ref[...], w3_ref[...], preferred_element_type=jnp.float32)
        + b_ref[...]
    )


def _edge_mm(edge_attr, w3, b_msg):
    eb = 4000
    grid = N_EDGES // eb
    return pl.pallas_call(
        _edge_mm_body,
        grid=(grid,),
        in_specs=[
            pl.BlockSpec((eb, D_EDGE), lambda i: (i, 0)),
            pl.BlockSpec((D_EDGE, D), lambda i: (0, 0)),
            pl.BlockSpec((D,), lambda i: (0,)),
        ],
        out_specs=pl.BlockSpec((eb, D), lambda i: (i, 0)),
        out_shape=jax.ShapeDtypeStruct((N_EDGES, D), jnp.float32),
    )(edge_attr, w3, b_msg)


def _sc_edge_body(ab_hbm, c_hbm, src_hbm, dst_hbm, out_hbm, deg_hbm,
                  cidx, didx, rows_ab, rows_c, msg, deg_v, acc,
                  sem_ab, sem_c):
    cid = lax.axis_index("c")
    sid = lax.axis_index("s")

    zv = jnp.zeros((16,), jnp.float32)

    def zrow(i, _):
        for j in range(D // 16):
            msg[i, pl.ds(j * 16, 16)] = zv
        return 0
    lax.fori_loop(0, CH, zrow, 0)

    def zdeg(i, _):
        deg_v[pl.ds(i * 16, 16)] = zv
        return 0
    lax.fori_loop(0, N_PAD // 16, zdeg, 0)

    n_zblk = ACC_ROWS // CH
    for t in range(-(-n_zblk // NS)):
        zb = jnp.minimum(sid + t * NS, n_zblk - 1)
        pltpu.sync_copy(msg, acc.at[pl.ds(zb * CH, CH)])

    plsc.subcore_barrier()

    ones16 = jnp.ones((16,), jnp.float32)
    trash16 = jnp.full((16,), N_PAD - 1, jnp.int32)
    n_iters = -(-N_CHUNKS // NS)

    def chunk_fn(ci, _):
        chunk = sid + ci * NS
        valid = chunk < N_CHUNKS
        e0 = jnp.where(valid, chunk, 0) * CH
        pltpu.sync_copy(src_hbm.at[pl.ds(e0, CH)], cidx.at[pl.ds(0, CH)])
        pltpu.sync_copy(dst_hbm.at[pl.ds(e0, CH)], cidx.at[pl.ds(CH, CH)])
        for k in range(CH // 16):
            dv = cidx[pl.ds(CH + k * 16, 16)]
            didx[pl.ds(k * 16, 16)] = dv
            cidx[pl.ds(CH + k * 16, 16)] = dv + N_NODES
        pltpu.sync_copy(c_hbm.at[pl.ds(e0, CH)], rows_c)
        cp_ab = pltpu.async_copy(ab_hbm.at[cidx], rows_ab, sem_ab)
        cp_ab.wait()

        for k in range(CH // 16):
            dv = didx[pl.ds(k * 16, 16)]
            dv = jnp.where(valid, dv, trash16)
            plsc.addupdate_scatter(deg_v, [dv], ones16)
            dl = dv - cid * NH
            oob = (dl < 0) | (dl >= NH) | jnp.logical_not(valid)
            didx[pl.ds(k * 16, 16)] = jnp.where(oob, NH, dl)

        def row_fn(i, _):
            for j in range(D // 16):
                sl = pl.ds(j * 16, 16)
                msg[i, sl] = jnp.maximum(
                    rows_ab[i, sl] + rows_ab[i + CH, sl] + rows_c[i, sl],
                    0.0)
            return 0
        lax.fori_loop(0, CH, row_fn, 0)

        pltpu.sync_copy(msg, acc.at[didx], add=True)
        return 0

    lax.fori_loop(0, n_iters, chunk_fn, 0)

    plsc.subcore_barrier()

    tb = sid * ROWS_PER_TILE
    pltpu.sync_copy(acc.at[pl.ds(tb, ROWS_PER_TILE)],
                    out_hbm.at[cid, pl.ds(tb, ROWS_PER_TILE)])
    pltpu.sync_copy(deg_v, deg_hbm.at[cid * NS + sid])


def _sc_edge(ab, c, src, dst):
    mesh = plsc.VectorSubcoreMesh(core_axis_name="c", subcore_axis_name="s")
    fn = pl.kernel(
        _sc_edge_body,
        out_type=[
            jax.ShapeDtypeStruct((NC, ACC_ROWS, D), jnp.float32),
            jax.ShapeDtypeStruct((NC * NS, N_PAD), jnp.float32),
        ],
        mesh=mesh,
        scratch_types=[
            pltpu.VMEM((2 * CH,), jnp.int32),
            pltpu.VMEM((CH,), jnp.int32),
            pltpu.VMEM((2 * CH, D), jnp.float32),
            pltpu.VMEM((CH, D), jnp.float32),
            pltpu.VMEM((CH, D), jnp.float32),
            pltpu.VMEM((N_PAD,), jnp.float32),
            pltpu.VMEM_SHARED((ACC_ROWS, D), jnp.float32),
            pltpu.SemaphoreType.DMA,
            pltpu.SemaphoreType.DMA,
        ],
        compiler_params=pltpu.CompilerParams(needs_layout_passes=False),
    )
    return fn(ab, c, src, dst)


def _update_body(x_ref, parts_ref, degs_ref, wu_ref, bu_ref, g_ref, bt_ref,
                 o_ref):
    agg_sum = parts_ref[0]
    deg = 0.5 * jnp.sum(degs_ref[...], axis=0)
    agg = agg_sum / jnp.maximum(deg, 1.0)[:, None]
    xb = x_ref[...]
    wu = wu_ref[...]
    u = (jnp.dot(xb, wu[:D], preferred_element_type=jnp.float32)
         + jnp.dot(agg, wu[D:], preferred_element_type=jnp.float32)
         + bu_ref[...])
    h = xb + jnp.maximum(u, 0.0)
    mu = jnp.mean(h, axis=-1, keepdims=True)
    var = jnp.mean((h - mu) ** 2, axis=-1, keepdims=True)
    o_ref[...] = g_ref[...] * (h - mu) * lax.rsqrt(var + 1e-5) + bt_ref[...]


def _update(x, parts, degs, w_upd, b_upd, g, bt):
    bn = 1024
    grid = N_PAD // bn
    return pl.pallas_call(
        _update_body,
        grid=(grid,),
        in_specs=[
            pl.BlockSpec((bn, D), lambda i: (i, 0)),
            pl.BlockSpec((1, bn, D), lambda i: (i // 5, i % 5, 0)),
            pl.BlockSpec((NC * NS, bn), lambda i: (0, i)),
            pl.BlockSpec((2 * D, D), lambda i: (0, 0)),
            pl.BlockSpec((D,), lambda i: (0,)),
            pl.BlockSpec((D,), lambda i: (0,)),
            pl.BlockSpec((D,), lambda i: (0,)),
        ],
        out_specs=pl.BlockSpec((bn, D), lambda i: (i, 0)),
        out_shape=jax.ShapeDtypeStruct((N_NODES, D), jnp.float32),
    )(x, parts, degs, w_upd, b_upd, g, bt)


def kernel(x, edge_index, edge_attr, W_msg, b_msg, W_upd, b_upd,
           ln_gamma, ln_beta):
    src = edge_index[0].astype(jnp.int32)
    dst = edge_index[1].astype(jnp.int32)
    w1 = W_msg[:D]
    w2 = W_msg[D:2 * D]
    w3 = W_msg[2 * D:]
    ab = _node_mm(x, w1, w2)
    c = _edge_mm(edge_attr, w3, b_msg)
    parts, degs = _sc_edge(ab, c, src, dst)
    return _update(x, parts, degs, W_upd, b_upd, ln_gamma, ln_beta)


# speedup vs baseline: 2.0850x; 1.0392x over previous
import jax
import jax.numpy as jnp
from jax import lax
from jax.experimental import pallas as pl
from jax.experimental.pallas import tpu as pltpu
from jax.experimental.pallas import tpu_sc as plsc

N_NODES = 10000
N_EDGES = 320000
D = 128
DH = 64
D_EDGE = 16

NC = 2
NS = 16
CH = 64
N_CHUNKS = N_EDGES // CH
NH = 5120
ACC_ROWS = 5248
ROWS_PER_TILE = ACC_ROWS // NS
N_PAD = 10240


def _node_mm_body(x_ref, w_ref, ab_ref):
    ab_ref[...] = jnp.dot(x_ref[...], w_ref[0],
                          preferred_element_type=jnp.float32)


def _node_mm(x, w1, w2):
    bn = 1000
    nblk = N_NODES // bn
    w12 = jnp.stack([w1, w2])
    return pl.pallas_call(
        _node_mm_body,
        grid=(2 * nblk,),
        in_specs=[
            pl.BlockSpec((bn, D), lambda i: (i % nblk, 0)),
            pl.BlockSpec((1, D, D), lambda i: (i // nblk, 0, 0)),
        ],
        out_specs=pl.BlockSpec((bn, D), lambda i: (i, 0)),
        out_shape=jax.ShapeDtypeStruct((2 * N_NODES, D), jnp.float32),
    )(x, w12)


def _edge_mm_body(e_ref, w3_ref, b_ref, c_ref):
    c_ref[...] = (
        jnp.dot(e_ref[...], w3_ref[...], preferred_element_type=jnp.float32)
        + b_ref[...]
    )


def _edge_mm(edge_attr, w3, b_msg):
    eb = 4000
    grid = N_EDGES // eb
    return pl.pallas_call(
        _edge_mm_body,
        grid=(grid,),
        in_specs=[
            pl.BlockSpec((eb, D_EDGE), lambda i: (i, 0)),
            pl.BlockSpec((D_EDGE, D), lambda i: (0, 0)),
            pl.BlockSpec((D,), lambda i: (0,)),
        ],
        out_specs=pl.BlockSpec((eb, D), lambda i: (i, 0)),
        out_shape=jax.ShapeDtypeStruct((N_EDGES, D), jnp.float32),
    )(edge_attr, w3, b_msg)


def _sc_edge_body(ab_hbm, c_hbm, src_hbm, dst_hbm, out_hbm, deg_hbm,
                  cidxf, didxgf, didxl2, rows_ab2, rows_c2, msg, deg_v, acc,
                  sem_g, sem_c, sem_i):
    cid = lax.axis_index("c")
    sid = lax.axis_index("s")

    zv = jnp.zeros((16,), jnp.float32)

    def zrow(i, _):
        for j in range(D // 16):
            msg[i, pl.ds(j * 16, 16)] = zv
        return 0
    lax.fori_loop(0, CH, zrow, 0)

    def zdeg(i, _):
        deg_v[pl.ds(i * 16, 16)] = zv
        return 0
    lax.fori_loop(0, N_PAD // 16, zdeg, 0)

    n_zblk = ACC_ROWS // CH
    for t in range(-(-n_zblk // NS)):
        zb = jnp.minimum(sid + t * NS, n_zblk - 1)
        pltpu.sync_copy(msg, acc.at[pl.ds(zb * CH, CH)])

    plsc.subcore_barrier()

    ones16 = jnp.ones((16,), jnp.float32)
    trash16 = jnp.full((16,), N_PAD - 1, jnp.int32)
    n_iters = -(-N_CHUNKS // NS)

    def e_of(ch):
        return jnp.where(ch < N_CHUNKS, ch, 0) * CH


    def fix(ch, ps):
        valid = ch < N_CHUNKS
        for k in range(CH // 16):
            dvg = didxgf[pl.ds(ps * CH + k * 16, 16)]
            cidxf[pl.ds(ps * 2 * CH + CH + k * 16, 16)] = dvg + N_NODES
            dvd = jnp.where(valid, dvg, trash16)
            plsc.addupdate_scatter(deg_v, [dvd], ones16)
            dl = dvg - cid * NH
            oob = (dl < 0) | (dl >= NH) | jnp.logical_not(valid)
            didxl2[ps, pl.ds(k * 16, 16)] = jnp.where(oob, NH, dl)

    def issue_idx(ch, ps):
        e0 = e_of(ch)
        pltpu.async_copy(src_hbm.at[pl.ds(e0, CH)],
                         cidxf.at[pl.ds(ps * 2 * CH, CH)], sem_i)
        pltpu.async_copy(dst_hbm.at[pl.ds(e0, CH)],
                         didxgf.at[pl.ds(ps * CH, CH)], sem_i)

    def drain_idx(ps):
        pltpu.make_async_copy(src_hbm.at[pl.ds(0, CH)],
                              cidxf.at[pl.ds(ps * 2 * CH, CH)], sem_i).wait()
        pltpu.make_async_copy(dst_hbm.at[pl.ds(0, CH)],
                              didxgf.at[pl.ds(ps * CH, CH)], sem_i).wait()

    def issue_gc(ch, ps):
        e0 = e_of(ch)
        pltpu.async_copy(ab_hbm.at[cidxf.at[pl.ds(ps * 2 * CH, 2 * CH)]],
                         rows_ab2.at[pl.ds(ps * 2 * CH, 2 * CH)], sem_g)
        pltpu.async_copy(c_hbm.at[pl.ds(e0, CH)],
                         rows_c2.at[pl.ds(ps * CH, CH)], sem_c)

    def drain_gc(ps):
        pltpu.make_async_copy(ab_hbm.at[pl.ds(0, 2 * CH)],
                              rows_ab2.at[pl.ds(ps * 2 * CH, 2 * CH)],
                              sem_g).wait()
        pltpu.make_async_copy(c_hbm.at[pl.ds(0, CH)],
                              rows_c2.at[pl.ds(ps * CH, CH)], sem_c).wait()

    ch0 = sid
    e0 = e_of(ch0)
    pltpu.sync_copy(src_hbm.at[pl.ds(e0, CH)], cidxf.at[pl.ds(0, CH)])
    pltpu.sync_copy(dst_hbm.at[pl.ds(e0, CH)], didxgf.at[pl.ds(0, CH)])
    fix(ch0, 0)
    issue_gc(ch0, 0)
    issue_idx(ch0 + NS, 1)

    def chunk_fn(ci, _):
        p = lax.rem(ci, 2)
        pn = 1 - p
        ch_cur = sid + ci * NS
        drain_idx(pn)
        fix(ch_cur + NS, pn)
        drain_gc(p)
        issue_gc(ch_cur + NS, pn)
        issue_idx(ch_cur + 2 * NS, p)

        poff = p * 2 * CH
        pcoff = p * CH

        def row_fn(i, _):
            for j in range(D // 16):
                sl = pl.ds(j * 16, 16)
                msg[i, sl] = jnp.maximum(
                    rows_ab2[poff + i, sl] + rows_ab2[poff + CH + i, sl]
                    + rows_c2[pcoff + i, sl], 0.0)
            return 0
        lax.fori_loop(0, CH, row_fn, 0)

        pltpu.sync_copy(msg, acc.at[didxl2.at[p]], add=True)
        return 0

    lax.fori_loop(0, n_iters, chunk_fn, 0)

    drain_gc(n_iters % 2)
    drain_idx((n_iters - 1) % 2)

    plsc.subcore_barrier()

    tb = sid * ROWS_PER_TILE
    pltpu.sync_copy(acc.at[pl.ds(tb, ROWS_PER_TILE)],
                    out_hbm.at[cid, pl.ds(tb, ROWS_PER_TILE)])
    pltpu.sync_copy(deg_v, deg_hbm.at[cid * NS + sid])


def _sc_edge(ab, c, src, dst):
    mesh = plsc.VectorSubcoreMesh(core_axis_name="c", subcore_axis_name="s")
    fn = pl.kernel(
        _sc_edge_body,
        out_type=[
            jax.ShapeDtypeStruct((NC, ACC_ROWS, D), jnp.float32),
            jax.ShapeDtypeStruct((NC * NS, N_PAD), jnp.float32),
        ],
        mesh=mesh,
        scratch_types=[
            pltpu.VMEM((2 * 2 * CH,), jnp.int32),
            pltpu.VMEM((2 * CH,), jnp.int32),
            pltpu.VMEM((2, CH), jnp.int32),
            pltpu.VMEM((2 * 2 * CH, D), jnp.float32),
            pltpu.VMEM((2 * CH, D), jnp.float32),
            pltpu.VMEM((CH, D), jnp.float32),
            pltpu.VMEM((N_PAD,), jnp.float32),
            pltpu.VMEM_SHARED((ACC_ROWS, D), jnp.float32),
            pltpu.SemaphoreType.DMA,
            pltpu.SemaphoreType.DMA,
            pltpu.SemaphoreType.DMA,
        ],
        compiler_params=pltpu.CompilerParams(needs_layout_passes=False),
    )
    return fn(ab, c, src, dst)


def _update_body(x_ref, parts_ref, degs_ref, wu_ref, bu_ref, g_ref, bt_ref,
                 o_ref):
    agg_sum = parts_ref[0]
    deg = 0.5 * jnp.sum(degs_ref[...], axis=0)
    agg = agg_sum / jnp.maximum(deg, 1.0)[:, None]
    xb = x_ref[...]
    wu = wu_ref[...]
    u = (jnp.dot(xb, wu[:D], preferred_element_type=jnp.float32)
         + jnp.dot(agg, wu[D:], preferred_element_type=jnp.float32)
         + bu_ref[...])
    h = xb + jnp.maximum(u, 0.0)
    mu = jnp.mean(h, axis=-1, keepdims=True)
    var = jnp.mean((h - mu) ** 2, axis=-1, keepdims=True)
    o_ref[...] = g_ref[...] * (h - mu) * lax.rsqrt(var + 1e-5) + bt_ref[...]


def _update(x, parts, degs, w_upd, b_upd, g, bt):
    bn = 1024
    grid = N_PAD // bn
    return pl.pallas_call(
        _update_body,
        grid=(grid,),
        in_specs=[
            pl.BlockSpec((bn, D), lambda i: (i, 0)),
            pl.BlockSpec((1, bn, D), lambda i: (i // 5, i % 5, 0)),
            pl.BlockSpec((NC * NS, bn), lambda i: (0, i)),
            pl.BlockSpec((2 * D, D), lambda i: (0, 0)),
            pl.BlockSpec((D,), lambda i: (0,)),
            pl.BlockSpec((D,), lambda i: (0,)),
            pl.BlockSpec((D,), lambda i: (0,)),
        ],
        out_specs=pl.BlockSpec((bn, D), lambda i: (i, 0)),
        out_shape=jax.ShapeDtypeStruct((N_NODES, D), jnp.float32),
    )(x, parts, degs, w_upd, b_upd, g, bt)


def kernel(x, edge_index, edge_attr, W_msg, b_msg, W_upd, b_upd,
           ln_gamma, ln_beta):
    src = edge_index[0].astype(jnp.int32)
    dst = edge_index[1].astype(jnp.int32)
    w1 = W_msg[:D]
    w2 = W_msg[D:2 * D]
    w3 = W_msg[2 * D:]
    ab = _node_mm(x, w1, w2)
    c = _edge_mm(edge_attr, w3, b_msg)
    parts, degs = _sc_edge(ab, c, src, dst)
    return _update(x, parts, degs, W_upd, b_upd, ln_gamma, ln_beta)


# speedup vs baseline: 2.1976x; 1.0540x over previous
import jax
import jax.numpy as jnp
from jax import lax
from jax.experimental import pallas as pl
from jax.experimental.pallas import tpu as pltpu
from jax.experimental.pallas import tpu_sc as plsc

N_NODES = 10000
N_EDGES = 320000
D = 128
DH = 64
D_EDGE = 16

NC = 2
NS = 16
CH = 64
N_CHUNKS = N_EDGES // CH
NH = 5120
ACC_ROWS = 5248
ROWS_PER_TILE = ACC_ROWS // NS
N_PAD = 10240


def _node_mm_body(x_ref, w_ref, ab_ref):
    ab_ref[...] = jnp.dot(x_ref[...], w_ref[0],
                          preferred_element_type=jnp.float32)


def _node_mm(x, w1, w2):
    bn = 1000
    nblk = N_NODES // bn
    w12 = jnp.stack([w1, w2])
    return pl.pallas_call(
        _node_mm_body,
        grid=(2 * nblk,),
        in_specs=[
            pl.BlockSpec((bn, D), lambda i: (i % nblk, 0)),
            pl.BlockSpec((1, D, D), lambda i: (i // nblk, 0, 0)),
        ],
        out_specs=pl.BlockSpec((bn, D), lambda i: (i, 0)),
        out_shape=jax.ShapeDtypeStruct((2 * N_NODES, D), jnp.float32),
    )(x, w12)


def _edge_mm_body(e_ref, w3_ref, b_ref, c_ref):
    c_ref[...] = (
        jnp.dot(e_ref[...], w3_ref[...], preferred_element_type=jnp.float32)
        + b_ref[...]
    )


def _edge_mm(edge_attr, w3, b_msg):
    eb = 4000
    grid = N_EDGES // eb
    return pl.pallas_call(
        _edge_mm_body,
        grid=(grid,),
        in_specs=[
            pl.BlockSpec((eb, D_EDGE), lambda i: (i, 0)),
            pl.BlockSpec((D_EDGE, D), lambda i: (0, 0)),
            pl.BlockSpec((D,), lambda i: (0,)),
        ],
        out_specs=pl.BlockSpec((eb, D), lambda i: (i, 0)),
        out_shape=jax.ShapeDtypeStruct((N_EDGES, D), jnp.float32),
    )(edge_attr, w3, b_msg)


def _sc_edge_body(ab_hbm, c_hbm, src_hbm, dst_hbm, out_hbm, deg_hbm,
                  cidxf, didxgf, didxl2, rows_ab2, rows_c2, msg, deg_v, acc,
                  sem_g, sem_c, sem_i):
    cid = lax.axis_index("c")
    sid = lax.axis_index("s")

    zv = jnp.zeros((16,), jnp.float32)

    def zrow(i, _):
        for j in range(D // 16):
            msg[i, pl.ds(j * 16, 16)] = zv
        return 0
    lax.fori_loop(0, CH, zrow, 0)

    def zdeg(i, _):
        deg_v[pl.ds(i * 16, 16)] = zv
        return 0
    lax.fori_loop(0, N_PAD // 16, zdeg, 0)

    n_zblk = ACC_ROWS // CH
    for t in range(-(-n_zblk // NS)):
        zb = jnp.minimum(sid + t * NS, n_zblk - 1)
        pltpu.sync_copy(msg, acc.at[pl.ds(zb * CH, CH)])

    plsc.subcore_barrier()

    ones16 = jnp.ones((16,), jnp.float32)
    trash16 = jnp.full((16,), N_PAD - 1, jnp.int32)
    n_iters = -(-N_CHUNKS // NS)

    def e_of(ch):
        return jnp.where(ch < N_CHUNKS, ch, 0) * CH


    def fix(ch, ps):
        valid = ch < N_CHUNKS
        for k in range(CH // 16):
            dvg = didxgf[pl.ds(ps * CH + k * 16, 16)]
            cidxf[pl.ds(ps * 2 * CH + CH + k * 16, 16)] = dvg + N_NODES
            dvd = jnp.where(valid, dvg, trash16)
            plsc.addupdate_scatter(deg_v, [dvd], ones16)
            dl = dvg - cid * NH
            oob = (dl < 0) | (dl >= NH) | jnp.logical_not(valid)
            didxl2[ps, pl.ds(k * 16, 16)] = jnp.where(oob, NH, dl)

    def issue_idx(ch, ps):
        e0 = e_of(ch)
        pltpu.async_copy(src_hbm.at[pl.ds(e0, CH)],
                         cidxf.at[pl.ds(ps * 2 * CH, CH)], sem_i)
        pltpu.async_copy(dst_hbm.at[pl.ds(e0, CH)],
                         didxgf.at[pl.ds(ps * CH, CH)], sem_i)

    def drain_idx(ps):
        pltpu.make_async_copy(src_hbm.at[pl.ds(0, CH)],
                              cidxf.at[pl.ds(ps * 2 * CH, CH)], sem_i).wait()
        pltpu.make_async_copy(dst_hbm.at[pl.ds(0, CH)],
                              didxgf.at[pl.ds(ps * CH, CH)], sem_i).wait()

    def issue_gc(ch, ps):
        e0 = e_of(ch)
        pltpu.async_copy(ab_hbm.at[cidxf.at[pl.ds(ps * 2 * CH, 2 * CH)]],
                         rows_ab2.at[pl.ds(ps * 2 * CH, 2 * CH)], sem_g)
        pltpu.async_copy(c_hbm.at[pl.ds(e0, CH)],
                         rows_c2.at[pl.ds(ps * CH, CH)], sem_c)

    def drain_gc(ps):
        pltpu.make_async_copy(ab_hbm.at[pl.ds(0, 2 * CH)],
                              rows_ab2.at[pl.ds(ps * 2 * CH, 2 * CH)],
                              sem_g).wait()
        pltpu.make_async_copy(c_hbm.at[pl.ds(0, CH)],
                              rows_c2.at[pl.ds(ps * CH, CH)], sem_c).wait()

    ch0 = sid
    e0 = e_of(ch0)
    pltpu.sync_copy(src_hbm.at[pl.ds(e0, CH)], cidxf.at[pl.ds(0, CH)])
    pltpu.sync_copy(dst_hbm.at[pl.ds(e0, CH)], didxgf.at[pl.ds(0, CH)])
    fix(ch0, 0)
    issue_gc(ch0, 0)
    issue_idx(ch0 + NS, 1)

    def chunk_fn(ci, _):
        p = lax.rem(ci, 2)
        pn = 1 - p
        ch_cur = sid + ci * NS
        drain_idx(pn)
        fix(ch_cur + NS, pn)
        drain_gc(p)
        issue_gc(ch_cur + NS, pn)
        issue_idx(ch_cur + 2 * NS, p)

        poff = p * 2 * CH
        pcoff = p * CH

        RU = 8

        def row_fn(i, _):
            r0 = i * RU
            for r in range(RU):
                for j in range(D // 16):
                    sl = pl.ds(j * 16, 16)
                    msg[r0 + r, sl] = jnp.maximum(
                        rows_ab2[poff + r0 + r, sl]
                        + rows_ab2[poff + CH + r0 + r, sl]
                        + rows_c2[pcoff + r0 + r, sl], 0.0)
            return 0
        lax.fori_loop(0, CH // RU, row_fn, 0)

        pltpu.sync_copy(msg, acc.at[didxl2.at[p]], add=True)
        return 0

    lax.fori_loop(0, n_iters, chunk_fn, 0)

    drain_gc(n_iters % 2)
    drain_idx((n_iters - 1) % 2)

    plsc.subcore_barrier()

    tb = sid * ROWS_PER_TILE
    pltpu.sync_copy(acc.at[pl.ds(tb, ROWS_PER_TILE)],
                    out_hbm.at[cid, pl.ds(tb, ROWS_PER_TILE)])
    pltpu.sync_copy(deg_v, deg_hbm.at[cid * NS + sid])


def _sc_edge(ab, c, src, dst):
    mesh = plsc.VectorSubcoreMesh(core_axis_name="c", subcore_axis_name="s")
    fn = pl.kernel(
        _sc_edge_body,
        out_type=[
            jax.ShapeDtypeStruct((NC, ACC_ROWS, D), jnp.float32),
            jax.ShapeDtypeStruct((NC * NS, N_PAD), jnp.float32),
        ],
        mesh=mesh,
        scratch_types=[
            pltpu.VMEM((2 * 2 * CH,), jnp.int32),
            pltpu.VMEM((2 * CH,), jnp.int32),
            pltpu.VMEM((2, CH), jnp.int32),
            pltpu.VMEM((2 * 2 * CH, D), jnp.float32),
            pltpu.VMEM((2 * CH, D), jnp.float32),
            pltpu.VMEM((CH, D), jnp.float32),
            pltpu.VMEM((N_PAD,), jnp.float32),
            pltpu.VMEM_SHARED((ACC_ROWS, D), jnp.float32),
            pltpu.SemaphoreType.DMA,
            pltpu.SemaphoreType.DMA,
            pltpu.SemaphoreType.DMA,
        ],
        compiler_params=pltpu.CompilerParams(needs_layout_passes=False),
    )
    return fn(ab, c, src, dst)


def _update_body(x_ref, parts_ref, degs_ref, wu_ref, bu_ref, g_ref, bt_ref,
                 o_ref):
    agg_sum = parts_ref[0]
    deg = 0.5 * jnp.sum(degs_ref[...], axis=0)
    agg = agg_sum / jnp.maximum(deg, 1.0)[:, None]
    xb = x_ref[...]
    wu = wu_ref[...]
    u = (jnp.dot(xb, wu[:D], preferred_element_type=jnp.float32)
         + jnp.dot(agg, wu[D:], preferred_element_type=jnp.float32)
         + bu_ref[...])
    h = xb + jnp.maximum(u, 0.0)
    mu = jnp.mean(h, axis=-1, keepdims=True)
    var = jnp.mean((h - mu) ** 2, axis=-1, keepdims=True)
    o_ref[...] = g_ref[...] * (h - mu) * lax.rsqrt(var + 1e-5) + bt_ref[...]


def _update(x, parts, degs, w_upd, b_upd, g, bt):
    bn = 1024
    grid = N_PAD // bn
    return pl.pallas_call(
        _update_body,
        grid=(grid,),
        in_specs=[
            pl.BlockSpec((bn, D), lambda i: (i, 0)),
            pl.BlockSpec((1, bn, D), lambda i: (i // 5, i % 5, 0)),
            pl.BlockSpec((NC * NS, bn), lambda i: (0, i)),
            pl.BlockSpec((2 * D, D), lambda i: (0, 0)),
            pl.BlockSpec((D,), lambda i: (0,)),
            pl.BlockSpec((D,), lambda i: (0,)),
            pl.BlockSpec((D,), lambda i: (0,)),
        ],
        out_specs=pl.BlockSpec((bn, D), lambda i: (i, 0)),
        out_shape=jax.ShapeDtypeStruct((N_NODES, D), jnp.float32),
    )(x, parts, degs, w_upd, b_upd, g, bt)


def kernel(x, edge_index, edge_attr, W_msg, b_msg, W_upd, b_upd,
           ln_gamma, ln_beta):
    src = edge_index[0].astype(jnp.int32)
    dst = edge_index[1].astype(jnp.int32)
    w1 = W_msg[:D]
    w2 = W_msg[D:2 * D]
    w3 = W_msg[2 * D:]
    ab = _node_mm(x, w1, w2)
    c = _edge_mm(edge_attr, w3, b_msg)
    parts, degs = _sc_edge(ab, c, src, dst)
    return _update(x, parts, degs, W_upd, b_upd, ln_gamma, ln_beta)
